# baseline (speedup 1.0000x reference)
import functools

import jax
import jax.numpy as jnp
from jax import lax
from jax.experimental import pallas as pl
from jax.experimental.pallas import tpu as pltpu
from jax.experimental.pallas import tpu_sc as plsc
from jax._src.pallas import mpmd

_N = 512
_D = 128
_K = 10
_T = 2 * _K + 1
_NC = 2
_RPC = _N // _NC
_LO = _N - 11
_S = 64
_RF = _LO + _T
_EARLY = _K + 1

_NW = 32
_RPW = 2
_TEC0 = 224
_TECN = _NW * _RPW
_WIN = _N + _RPW - 1
_CH = 128
_NCH = -(-_WIN // _CH)
_SCR = 32 + _EARLY


def _index_grid():
    w = jnp.arange(_NW, dtype=jnp.int32)[:, None]
    t = jnp.arange(_NCH * _CH, dtype=jnp.int32)[None, :]
    return jnp.clip(t - 215 - 2 * w, 0, _T - 1).reshape(_NW, _NCH, _CH)


def _scs_fn(table_hbm, idx_hbm, out_hbm, b_sh):
    del idx_hbm

    def body(sem, fsem):
        cid = lax.axis_index("c")
        r0 = cid * _RPC
        lscr = out_hbm.at[r0 + _SCR]
        rscr = out_hbm.at[r0 + _SCR + 1]
        pltpu.sync_copy(table_hbm, b_sh.at[pl.ds(_LO, _T)])
        cs = []
        for k in range(8):
            cs.append(
                pltpu.async_copy(
                    b_sh.at[pl.ds(_LO, 1)], b_sh.at[pl.ds(k, 1)], fsem
                )
            )
            cs.append(
                pltpu.async_copy(
                    b_sh.at[pl.ds(_LO + _T - 1, 1)],
                    b_sh.at[pl.ds(_RF + k, 1)],
                    fsem,
                )
            )
        for c in cs:
            c.wait()
        cs = []
        for k in range(_S // 8):
            cs.append(
                pltpu.async_copy(
                    b_sh.at[pl.ds(0, 8)], lscr.at[pl.ds(8 * k, 8)], fsem
                )
            )
            cs.append(
                pltpu.async_copy(
                    b_sh.at[pl.ds(_RF, 8)], rscr.at[pl.ds(8 * k, 8)], fsem
                )
            )
        for c in cs:
            c.wait()

        def read_right_flank():
            return [
                pltpu.async_copy(
                    rscr.at[pl.ds(0, _S)],
                    b_sh.at[pl.ds(_RF + 8 + _S * k, _S)],
                    fsem,
                )
                for k in range(8)
            ]

        def read_left_flank():
            cs = [
                pltpu.async_copy(
                    lscr.at[pl.ds(0, _S)], b_sh.at[pl.ds(8 + _S * k, _S)], fsem
                )
                for k in range(7)
            ]
            cs.append(
                pltpu.async_copy(
                    lscr.at[pl.ds(0, 32)], b_sh.at[pl.ds(8 + 7 * _S, 32)], fsem
                )
            )
            cs.append(
                pltpu.async_copy(
                    b_sh.at[pl.ds(0, 8)], b_sh.at[pl.ds(488, 8)], fsem
                )
            )
            cs.append(
                pltpu.async_copy(
                    b_sh.at[pl.ds(0, 5)], b_sh.at[pl.ds(496, 5)], fsem
                )
            )
            return cs

        def issue_rows(lo, hi):
            def issue(i, carry):
                row = r0 + i
                pltpu.async_copy(
                    b_sh.at[pl.ds(_N - 1 - row, _N)], out_hbm.at[row], sem
                )
                return carry

            lax.fori_loop(lo, hi, issue, 0)

        @pl.when(cid == 0)
        def _():
            for c in read_right_flank():
                c.wait()
            issue_rows(0, _EARLY)
            for c in read_left_flank():
                c.wait()
            issue_rows(_EARLY, _TEC0)

        @pl.when(cid == 1)
        def _():
            for c in read_left_flank():
                c.wait()
            issue_rows(_RPC - _EARLY, _RPC)
            for c in read_right_flank():
                c.wait()
            issue_rows(_TEC0 + _TECN - _RPC, _RPC - _EARLY)

        def drain(i, carry):
            pltpu.make_async_copy(
                out_hbm.at[0], b_sh.at[pl.ds(0, _N)], sem
            ).wait()
            return carry

        lax.fori_loop(0, _TEC0, drain, 0)

    pl.run_scoped(body, pltpu.SemaphoreType.DMA, pltpu.SemaphoreType.DMA)


def _tec_fn(table_hbm, idx_hbm, out_hbm, b_sh):
    del idx_hbm, b_sh
    wid = lax.axis_index("s") * _NC + lax.axis_index("c")

    def body(win_v, sem):
        gathers = []
        for g in range(_NCH * _CH // 16):
            idx = lax.iota(jnp.int32, 16) + (16 * g - 215 - 2 * wid)
            idx = lax.max(idx, jnp.int32(0))
            idx = lax.min(idx, jnp.int32(_T - 1))
            gathers.append(
                pltpu.async_copy(
                    table_hbm.at[idx], win_v.at[pl.ds(16 * g, 16)], sem
                )
            )
        for gth in gathers:
            gth.wait()
        r1 = _TEC0 + _RPW * wid
        writes = [
            pltpu.async_copy(
                win_v.at[pl.ds(_RPW - 1 - p, _N)], out_hbm.at[r1 + p], sem
            )
            for p in range(_RPW)
        ]
        for wr in writes:
            wr.wait()

    pl.run_scoped(
        body,
        pltpu.VMEM((_NCH * _CH, _D), jnp.float32),
        pltpu.SemaphoreType.DMA,
    )


_hybrid = mpmd.mpmd_map(
    [
        (plsc.ScalarSubcoreMesh(axis_name="c", num_cores=_NC), _scs_fn),
        (
            plsc.VectorSubcoreMesh(
                core_axis_name="c", subcore_axis_name="s"
            ),
            _tec_fn,
        ),
    ],
    out_types=jax.ShapeDtypeStruct((_N, _N, _D), jnp.float32),
    scratch_types=[
        pltpu.VMEM_SHARED((_RF + 8 + 8 * _S, _D), jnp.float32),
    ],
)


def kernel(num_nodes, table):
    del num_nodes
    return _hybrid(table, _index_grid())


# speedup vs baseline: 8.3967x; 8.3967x over previous
import functools

import jax
import jax.numpy as jnp
from jax import lax
from jax.experimental import pallas as pl
from jax.experimental.pallas import tpu as pltpu
from jax.experimental.pallas import tpu_sc as plsc

_N = 512
_D = 128
_K = 10
_T = 2 * _K + 1
_NC = 2
_RPC = _N // _NC
_LO = _N - 11
_S = 64
_RF = _LO + _T
_EARLY = _K + 1


@functools.partial(
    pl.kernel,
    out_type=jax.ShapeDtypeStruct((_N, _N, _D), jnp.float32),
    mesh=plsc.ScalarSubcoreMesh(axis_name="c", num_cores=_NC),
    scratch_types=[
        pltpu.VMEM_SHARED((_RF + 8 + 8 * _S, _D), jnp.float32),
        pltpu.SemaphoreType.DMA,
        pltpu.SemaphoreType.DMA,
    ],
)
def _sc_band_fill(table_hbm, out_hbm, b_sh, sem, fsem):
    cid = lax.axis_index("c")
    r0 = cid * _RPC
    lscr = out_hbm.at[r0 + _EARLY]
    rscr = out_hbm.at[r0 + _EARLY + 1]
    pltpu.sync_copy(table_hbm, b_sh.at[pl.ds(_LO, _T)])
    cs = []
    for k in range(8):
        cs.append(
            pltpu.async_copy(b_sh.at[pl.ds(_LO, 1)], b_sh.at[pl.ds(k, 1)], fsem)
        )
        cs.append(
            pltpu.async_copy(
                b_sh.at[pl.ds(_LO + _T - 1, 1)], b_sh.at[pl.ds(_RF + k, 1)], fsem
            )
        )
    for c in cs:
        c.wait()
    cs = []
    for k in range(_S // 8):
        cs.append(
            pltpu.async_copy(b_sh.at[pl.ds(0, 8)], lscr.at[pl.ds(8 * k, 8)], fsem)
        )
        cs.append(
            pltpu.async_copy(
                b_sh.at[pl.ds(_RF, 8)], rscr.at[pl.ds(8 * k, 8)], fsem
            )
        )
    for c in cs:
        c.wait()

    def read_right_flank():
        cs = [
            pltpu.async_copy(
                rscr.at[pl.ds(0, _S)], b_sh.at[pl.ds(_RF + 8 + _S * k, _S)], fsem
            )
            for k in range(8)
        ]
        return cs

    def read_left_flank():
        cs = [
            pltpu.async_copy(
                lscr.at[pl.ds(0, _S)], b_sh.at[pl.ds(8 + _S * k, _S)], fsem
            )
            for k in range(7)
        ]
        cs.append(
            pltpu.async_copy(
                lscr.at[pl.ds(0, 32)], b_sh.at[pl.ds(8 + 7 * _S, 32)], fsem
            )
        )
        cs.append(
            pltpu.async_copy(b_sh.at[pl.ds(0, 8)], b_sh.at[pl.ds(488, 8)], fsem)
        )
        cs.append(
            pltpu.async_copy(b_sh.at[pl.ds(0, 5)], b_sh.at[pl.ds(496, 5)], fsem)
        )
        return cs

    def issue_rows(lo, hi):
        def issue(i, carry):
            row = r0 + i
            pltpu.async_copy(
                b_sh.at[pl.ds(_N - 1 - row, _N)], out_hbm.at[row], sem
            )
            return carry

        lax.fori_loop(lo, hi, issue, 0)

    @pl.when(cid == 0)
    def _():
        for c in read_right_flank():
            c.wait()
        issue_rows(0, _EARLY)
        for c in read_left_flank():
            c.wait()
        issue_rows(_EARLY, _RPC)

    @pl.when(cid == 1)
    def _():
        for c in read_left_flank():
            c.wait()
        issue_rows(_RPC - _EARLY, _RPC)
        for c in read_right_flank():
            c.wait()
        issue_rows(0, _RPC - _EARLY)

    def drain(i, carry):
        pltpu.make_async_copy(
            out_hbm.at[0], b_sh.at[pl.ds(0, _N)], sem
        ).wait()
        return carry

    lax.fori_loop(0, _RPC, drain, 0)


def kernel(num_nodes, table):
    del num_nodes
    return _sc_band_fill(table)
